# baseline (speedup 1.0000x reference)
import functools

import jax
import jax.numpy as jnp
from jax import lax
from jax.experimental import pallas as pl
from jax.experimental.pallas import tpu as pltpu
from jax.experimental.pallas import tpu_sc as plsc

BATCH = 16384
EMBED_DIM = 32
NUM_ROWS = 1000000
MAX_RATING = 5.0
MIN_RATING = 0.5

_info = plsc.get_sparse_core_info()
NC, NS, LANES = _info.num_cores, _info.num_subcores, _info.num_lanes
NW = NC * NS

SEG = 512
SEGS_T = 61
NSEG_MAX = 64
TAIL_OFF = 999936
CHUNK = 128
NVEC = BATCH // LANES

_mesh = plsc.VectorSubcoreMesh(core_axis_name="c", subcore_axis_name="s")

_k1_params = pltpu.CompilerParams(
    needs_layout_passes=False,
    use_tc_tiling_on_sc=True,
)


@functools.partial(
    pl.kernel,
    out_type=(
        jax.ShapeDtypeStruct((BATCH * EMBED_DIM,), jnp.float32),
        jax.ShapeDtypeStruct((BATCH * EMBED_DIM,), jnp.float32),
        jax.ShapeDtypeStruct((BATCH,), jnp.float32),
        jax.ShapeDtypeStruct((BATCH,), jnp.float32),
    ),
    mesh=_mesh,
    compiler_params=_k1_params,
    scratch_types=[
        pltpu.VMEM((BATCH + 16,), jnp.int32),
        pltpu.VMEM((BATCH + 16,), jnp.int32),
        pltpu.VMEM((BATCH + 16,), jnp.int32),
        pltpu.VMEM((BATCH + 16,), jnp.int32),
        pltpu.VMEM((96,), jnp.int32),
        pltpu.VMEM((96,), jnp.int32),
        pltpu.VMEM((96,), jnp.int32),
        pltpu.VMEM((2, 32, SEG), jnp.float32),
        pltpu.VMEM((CHUNK * EMBED_DIM,), jnp.float32),
        pltpu.VMEM((32, 128), jnp.int32),
        pltpu.VMEM((1, 128), jnp.int32),
        pltpu.VMEM((1, 128), jnp.int32),
        pltpu.VMEM((128,), jnp.float32),
        pltpu.SemaphoreType.DMA,
        pltpu.SemaphoreType.DMA,
        pltpu.SemaphoreType.DMA,
        pltpu.SemaphoreType.DMA,
    ],
)
def _k1(uidx_hbm, midx_hbm, ue_hbm, me_hbm, ub_hbm, mb_hbm,
        su_hbm, sm_hbm, sub_hbm, smb_hbm,
        idxv, hvun, hbun, hbs, hist, cum0, cur,
        ring, rowsf, offs2, bidx2, bofs2, bval,
        sems0, sems1, semsc, semb):
    wid = lax.axis_index("s") * NC + lax.axis_index("c")
    is_last = wid == NW - 1
    seg0 = wid * SEGS_T
    lo = seg0 * SEG
    nseg = jnp.where(is_last, SEGS_T + 1, SEGS_T)
    ntot = nseg + is_last.astype(jnp.int32)
    hi = jnp.where(is_last, NUM_ROWS, lo + SEGS_T * SEG)

    lane = jnp.arange(LANES, dtype=jnp.int32)
    zeros16 = jnp.zeros((LANES,), jnp.int32)
    lane0 = lane == 0
    ring_sems = [sems0, sems1]

    def sread(ref, h):
        return ref[pl.ds(h, 16)][0]

    def swrite1(ref, o, val):
        plsc.store_scatter(ref, [zeros16 + o], zeros16 + val, mask=lane0)

    def fetch(tbl, s, slot):
        off = pl.multiple_of((seg0 + s) * SEG, 128)
        is_tail = s >= nseg

        @pl.when(jnp.logical_not(is_tail))
        def _():
            pltpu.async_copy(tbl.at[:, pl.ds(off, SEG)],
                             ring.at[slot], ring_sems[slot])

        @pl.when(is_tail)
        def _():
            toff = pl.multiple_of(TAIL_OFF + 0 * s, 128)
            pltpu.async_copy(tbl.at[:, pl.ds(toff, 128)],
                             ring.at[slot, :, pl.ds(0, 128)],
                             ring_sems[slot])

    def drain(slot, wide):
        @pl.when(wide)
        def _():
            pltpu.make_async_copy(ue_hbm.at[:, pl.ds(0, SEG)],
                                  ring.at[slot], ring_sems[slot]).wait()

        @pl.when(jnp.logical_not(wide))
        def _():
            pltpu.make_async_copy(ue_hbm.at[:, pl.ds(0, 128)],
                                  ring.at[slot, :, pl.ds(0, 128)],
                                  ring_sems[slot]).wait()

    def flush(A, G, rows_out, bias_tbl, bias_out):
        def bld(k, carry):
            sl = pl.ds(k * 16, 16)
            act = (k * 16 + lane) < A
            v = idxv[pl.ds(G + k * 16, 16)]
            b = hbs[pl.ds(G + k * 16, 16)]
            bidx2[0, sl] = jnp.where(act, v + lo, 0)
            bofs2[0, sl] = jnp.where(act, b, -1)
            return carry
        lax.fori_loop(0, 8, bld, 0)

        cps = []
        for k in range(32):
            cps.append(pltpu.async_copy(
                rowsf.at[pl.ds(k * 128, 128)],
                rows_out.at[plsc.Indices(offs2.at[k], ignored_value=-1)],
                semsc))
        gb = pltpu.async_copy(bias_tbl.at[bidx2.at[0]], bval, semb)
        gb.wait()
        sb = pltpu.async_copy(
            bval, bias_out.at[plsc.Indices(bofs2.at[0], ignored_value=-1)],
            semb)
        for cp in cps:
            cp.wait()
        sb.wait()

    def reset_offs(_):
        def z(k, carry):
            for j in range(8):
                offs2[k, pl.ds(j * 16, 16)] = zeros16 - 1
            return carry
        lax.fori_loop(0, 32, z, 0)

    def process_table(idx_hbm, tbl_hbm, rows_out, bias_tbl, bias_out):
        pltpu.sync_copy(idx_hbm, idxv.at[pl.ds(0, BATCH)])

        def scan(k, cnt):
            i = idxv[pl.ds(k * 16, 16)]
            m = jnp.logical_and(i >= lo, i < hi)
            plsc.store_compressed(hvun.at[pl.ds(cnt, 16)], i - lo, mask=m)
            plsc.store_compressed(hbun.at[pl.ds(cnt, 16)],
                                  k * 16 + lane, mask=m)
            return cnt + plsc.all_reduce_population_count(m)[0]

        nh = lax.fori_loop(0, NVEC, scan, jnp.int32(0))

        def zcnt(k, carry):
            hist[pl.ds(k * 16, 16)] = zeros16
            return carry
        lax.fori_loop(0, 6, zcnt, 0)

        def hpass(h, carry):
            s = sread(hvun, h) >> 9
            swrite1(hist, s, sread(hist, s) + 1)
            return carry
        lax.fori_loop(0, nh, hpass, 0)

        def cpass(s, acc):
            swrite1(cum0, s, acc)
            swrite1(cur, s, acc)
            return acc + sread(hist, s)
        total = lax.fori_loop(0, NSEG_MAX + 1, cpass, jnp.int32(0))
        swrite1(cum0, NSEG_MAX + 1, total)

        def ppass(h, carry):
            v = sread(hvun, h)
            b = sread(hbun, h)
            s = v >> 9
            o = sread(cur, s)
            swrite1(cur, s, o + 1)
            swrite1(idxv, o, v)
            swrite1(hbs, o, b)
            return carry
        lax.fori_loop(0, nh, ppass, 0)

        reset_offs(0)
        fetch(tbl_hbm, jnp.int32(0), 0)
        fetch(tbl_hbm, jnp.int32(1), 1)

        def make_proc(slot, s):
            def proc(ag):
                A, G = ag
                drain(slot, s < nseg)
                st = sread(cum0, s)
                en = sread(cum0, s + 1)

                def chunk_cond(c):
                    _, _, pos = c
                    return pos < en

                def chunk_body(c):
                    A, G, pos = c
                    k = jnp.minimum(CHUNK - A, en - pos)

                    def ext(h, carry):
                        slotA = A + h
                        v = sread(idxv, pos + h)
                        b = sread(hbs, pos + h)
                        l = v - s * SEG
                        e = slotA * 32
                        for half in range(2):
                            g = plsc.load_gather(
                                ring.at[slot],
                                [lane + half * 16, zeros16 + l])
                            rowsf[pl.ds(e + half * 16, 16)] = g
                            plsc.store_scatter(
                                offs2,
                                [zeros16 + ((e + half * 16) >> 7),
                                 ((e + half * 16) & 127) + lane],
                                b * 32 + half * 16 + lane)
                        return carry
                    lax.fori_loop(0, k, ext, 0)
                    A2 = A + k

                    @pl.when(A2 == CHUNK)
                    def _():
                        flush(A2, G, rows_out, bias_tbl, bias_out)
                        reset_offs(0)

                    G2 = jnp.where(A2 == CHUNK, G + A2, G)
                    A3 = jnp.where(A2 == CHUNK, 0, A2)
                    return A3, G2, pos + k

                A, G, _ = lax.while_loop(chunk_cond, chunk_body, (A, G, st))

                @pl.when(s + 2 < ntot)
                def _():
                    fetch(tbl_hbm, s + 2, slot)
                return A, G
            return proc

        def seg_body(s, carry):
            return lax.cond(s % 2 == 0,
                            make_proc(0, s),
                            make_proc(1, s),
                            carry)

        A, G = lax.fori_loop(0, ntot, seg_body,
                             (jnp.int32(0), jnp.int32(0)))

        @pl.when(A > 0)
        def _():
            flush(A, G, rows_out, bias_tbl, bias_out)


    process_table(uidx_hbm, ue_hbm, su_hbm, ub_hbm, sub_hbm)
    process_table(midx_hbm, me_hbm, sm_hbm, mb_hbm, smb_hbm)


def _k2_body(su_ref, sm_ref, sub_ref, smb_ref, out_ref):
    dot = jnp.sum(su_ref[...] * sm_ref[...], axis=1)
    x = dot + sub_ref[...] + smb_ref[...]
    r = jax.nn.sigmoid(x)
    out_ref[...] = r * (MAX_RATING - MIN_RATING) + MIN_RATING


_K2_BLK = 2048


def _k2(su, sm, sub, smb):
    grid = (BATCH // _K2_BLK,)
    return pl.pallas_call(
        _k2_body,
        out_shape=jax.ShapeDtypeStruct((BATCH,), jnp.float32),
        grid=grid,
        in_specs=[
            pl.BlockSpec((_K2_BLK, EMBED_DIM), lambda i: (i, 0)),
            pl.BlockSpec((_K2_BLK, EMBED_DIM), lambda i: (i, 0)),
            pl.BlockSpec((_K2_BLK,), lambda i: (i,)),
            pl.BlockSpec((_K2_BLK,), lambda i: (i,)),
        ],
        out_specs=pl.BlockSpec((_K2_BLK,), lambda i: (i,)),
    )(su.reshape(BATCH, EMBED_DIM), sm.reshape(BATCH, EMBED_DIM), sub, smb)


def kernel(inputs, user_embedding, user_bias, movie_embedding, movie_bias):
    idx = inputs.astype(jnp.int32)
    su, sm, sub, smb = _k1(idx[:, 0], idx[:, 1],
                           user_embedding.T, movie_embedding.T,
                           user_bias.reshape(-1), movie_bias.reshape(-1))
    return _k2(su, sm, sub, smb)


# speedup vs baseline: 6.6819x; 6.6819x over previous
import functools

import jax
import jax.numpy as jnp
from jax import lax
from jax.experimental import pallas as pl
from jax.experimental.pallas import tpu as pltpu
from jax.experimental.pallas import tpu_sc as plsc

BATCH = 16384
EMBED_DIM = 32
NUM_ROWS = 1000000
MAX_RATING = 5.0
MIN_RATING = 0.5

_info = plsc.get_sparse_core_info()
NC, NS, LANES = _info.num_cores, _info.num_subcores, _info.num_lanes
NW = NC * NS

SEG = 512
SEGS_T = 61
NSEG_MAX = 64
TAIL_OFF = 999936
CHUNK = 128
CAP = 640
NVEC = BATCH // LANES

_mesh = plsc.VectorSubcoreMesh(core_axis_name="c", subcore_axis_name="s")

_k1_params = pltpu.CompilerParams(
    needs_layout_passes=False,
    use_tc_tiling_on_sc=True,
)


@functools.partial(
    pl.kernel,
    out_type=(
        jax.ShapeDtypeStruct((2 * BATCH * EMBED_DIM,), jnp.float32),
        jax.ShapeDtypeStruct((2 * BATCH * EMBED_DIM,), jnp.float32),
        jax.ShapeDtypeStruct((2 * BATCH,), jnp.float32),
        jax.ShapeDtypeStruct((2 * BATCH,), jnp.float32),
    ),
    mesh=_mesh,
    compiler_params=_k1_params,
    scratch_types=[
        pltpu.VMEM((BATCH + CAP + 16,), jnp.int32),
        pltpu.VMEM((192,), jnp.int32),
        pltpu.VMEM((192,), jnp.int32),
        pltpu.VMEM((2, 32, SEG), jnp.float32),
        pltpu.VMEM((CHUNK * EMBED_DIM,), jnp.float32),
        pltpu.VMEM((32, 128), jnp.int32),
        pltpu.VMEM((1, 128), jnp.int32),
        pltpu.VMEM((1, 128), jnp.int32),
        pltpu.VMEM((128,), jnp.float32),
        pltpu.VMEM_SHARED((BATCH * EMBED_DIM,), jnp.float32),
        pltpu.VMEM_SHARED((BATCH,), jnp.float32),
        pltpu.SMEM((CAP + 16,), jnp.int32),
        pltpu.SMEM((CAP + 16,), jnp.int32),
        pltpu.SMEM((72,), jnp.int32),
        pltpu.SMEM((72,), jnp.int32),
        pltpu.SMEM((72,), jnp.int32),
        pltpu.SemaphoreType.DMA,
        pltpu.SemaphoreType.DMA,
        pltpu.SemaphoreType.DMA,
        pltpu.SemaphoreType.DMA,
    ],
)
def _k1(uidx_hbm, midx_hbm, ue_hbm, me_hbm, ub_hbm, mb_hbm,
        su_hbm, sm_hbm, sub_hbm, smb_hbm,
        pkun, flushi, flushb, ring, rowsf, offs2, bidx2, bofs2, bval,
        srows, sbias,
        pks, pk2s, hists, cums, curs,
        sems0, sems1, semsc, semb):
    idxv = pkun
    wid = lax.axis_index("s") * NC + lax.axis_index("c")
    is_last = wid == NW - 1
    seg0 = wid * SEGS_T
    lo = seg0 * SEG
    nseg = jnp.where(is_last, SEGS_T + 1, SEGS_T)
    ntot = nseg + is_last.astype(jnp.int32)
    hi = jnp.where(is_last, NUM_ROWS, lo + SEGS_T * SEG)

    lane = jnp.arange(LANES, dtype=jnp.int32)
    zeros16 = jnp.zeros((LANES,), jnp.int32)
    lane0 = lane == 0
    ring_sems = [sems0, sems1]

    def sread(ref, h):
        return ref[pl.ds(h, 16)][0]

    def swrite1(ref, o, val):
        plsc.store_scatter(ref, [zeros16 + o], zeros16 + val, mask=lane0)

    def fetch(tbl, s, slot):
        off = pl.multiple_of((seg0 + s) * SEG, 128)
        is_tail = s >= nseg

        @pl.when(jnp.logical_not(is_tail))
        def _():
            pltpu.async_copy(tbl.at[:, pl.ds(off, SEG)],
                             ring.at[slot], ring_sems[slot])

        @pl.when(is_tail)
        def _():
            toff = pl.multiple_of(TAIL_OFF + 0 * s, 128)
            pltpu.async_copy(tbl.at[:, pl.ds(toff, 128)],
                             ring.at[slot, :, pl.ds(0, 128)],
                             ring_sems[slot])

    def drain(slot, wide):
        @pl.when(wide)
        def _():
            pltpu.make_async_copy(ue_hbm.at[:, pl.ds(0, SEG)],
                                  ring.at[slot], ring_sems[slot]).wait()

        @pl.when(jnp.logical_not(wide))
        def _():
            pltpu.make_async_copy(ue_hbm.at[:, pl.ds(0, 128)],
                                  ring.at[slot, :, pl.ds(0, 128)],
                                  ring_sems[slot]).wait()

    def flush(A, bias_tbl):
        def bld(k, carry):
            sl = pl.ds(k * 16, 16)
            act = (k * 16 + lane) < A
            v = flushi[sl]
            b = flushb[sl]
            bidx2[0, sl] = jnp.where(act, v, 0)
            bofs2[0, sl] = jnp.where(act, b, -1)
            return carry
        lax.fori_loop(0, 8, bld, 0)

        cps = []
        for k in range(32):
            cps.append(pltpu.async_copy(
                rowsf.at[pl.ds(k * 128, 128)],
                srows.at[plsc.Indices(offs2.at[k], ignored_value=-1)],
                semsc))
        gb = pltpu.async_copy(bias_tbl.at[bidx2.at[0]], bval, semb)
        gb.wait()
        sb = pltpu.async_copy(
            bval, sbias.at[plsc.Indices(bofs2.at[0], ignored_value=-1)],
            semb)
        for cp in cps:
            cp.wait()
        sb.wait()

    def reset_offs(_):
        def z(k, carry):
            for j in range(8):
                offs2[k, pl.ds(j * 16, 16)] = zeros16 - 1
            return carry
        lax.fori_loop(0, 32, z, 0)

    def process_table(idx_hbm, tbl_hbm, bias_tbl):
        pltpu.sync_copy(idx_hbm, idxv.at[pl.ds(0, BATCH)])

        def scan(k, cnt):
            i = idxv[pl.ds(k * 16, 16)]
            m = jnp.logical_and(i >= lo, i < hi)
            pk = (i - lo) * 16384 + (k * 16 + lane)
            plsc.store_compressed(pkun.at[pl.ds(cnt, 16)], pk, mask=m)
            return cnt + plsc.all_reduce_population_count(m)[0]

        nh = lax.fori_loop(0, NVEC, scan, jnp.int32(0))
        nbatch = (nh + CAP - 1) // CAP

        def batch_body(bi, carry):
            bn = jnp.minimum(jnp.int32(CAP), nh - bi * CAP)

            def spill(j, carry):
                vec = pkun[pl.ds(bi * CAP + j * 16, 16)]
                for ln in range(16):
                    pks[j * 16 + ln] = vec[ln]
                return carry
            lax.fori_loop(0, (bn + 15) // 16, spill, 0)

            def zcnt(s, carry):
                hists[s] = 0
                return carry
            lax.fori_loop(0, NSEG_MAX + 2, zcnt, 0)

            def hpass(h, carry):
                s = pks[h] >> 23
                hists[s] = hists[s] + 1
                return carry
            lax.fori_loop(0, bn, hpass, 0)

            def cpass(s, acc):
                cums[s] = acc
                curs[s] = acc
                return acc + hists[s]
            lax.fori_loop(0, NSEG_MAX + 2, cpass, jnp.int32(0))

            def ppass(h, carry):
                pk = pks[h]
                s = pk >> 23
                o = curs[s]
                curs[s] = o + 1
                pk2s[o] = pk
                return carry
            lax.fori_loop(0, bn, ppass, 0)

            reset_offs(0)
            fetch(tbl_hbm, jnp.int32(0), 0)
            fetch(tbl_hbm, jnp.int32(1), 1)

            def make_proc(slot, s):
                def proc(A):
                    drain(slot, s < nseg)
                    st = cums[s]
                    en = cums[s + 1]

                    def chunk_cond(c):
                        _, pos = c
                        return pos < en

                    def chunk_body(c):
                        A, pos = c
                        k = jnp.minimum(CHUNK - A, en - pos)

                        def ext(h, carry):
                            pk = pk2s[pos + h]
                            v = pk >> 14
                            b = pk & 16383
                            l = v - s * SEG
                            e = (A + h) * 32
                            swrite1(flushi, A + h, v + lo)
                            swrite1(flushb, A + h, b)
                            for half in range(2):
                                g = plsc.load_gather(
                                    ring.at[slot],
                                    [lane + half * 16, zeros16 + l])
                                rowsf[pl.ds(e + half * 16, 16)] = g
                                plsc.store_scatter(
                                    offs2,
                                    [zeros16 + ((e + half * 16) >> 7),
                                     ((e + half * 16) & 127) + lane],
                                    b * 32 + half * 16 + lane)
                            return carry
                        lax.fori_loop(0, k, ext, 0)
                        A2 = A + k

                        @pl.when(A2 == CHUNK)
                        def _():
                            flush(A2, bias_tbl)
                            reset_offs(0)

                        A3 = jnp.where(A2 == CHUNK, 0, A2)
                        return A3, pos + k

                    A, _ = lax.while_loop(chunk_cond, chunk_body, (A, st))

                    @pl.when(s + 2 < ntot)
                    def _():
                        fetch(tbl_hbm, s + 2, slot)
                    return A
                return proc

            def seg_body(s, A):
                return lax.cond(s % 2 == 0,
                                make_proc(0, s),
                                make_proc(1, s),
                                A)

            A = lax.fori_loop(0, ntot, seg_body, jnp.int32(0))

            @pl.when(A > 0)
            def _():
                flush(A, bias_tbl)
            return carry

        lax.fori_loop(0, nbatch, batch_body, 0)

    sid = lax.axis_index("s")
    core = lax.axis_index("c")
    RSL = BATCH * EMBED_DIM // NS
    BSL = BATCH // NS

    def zero_rowsf():
        zf = jnp.zeros((LANES,), jnp.float32)

        def z(j, carry):
            rowsf[pl.ds(j * 16, 16)] = zf
            return carry
        lax.fori_loop(0, CHUNK * EMBED_DIM // 16, z, 0)

    def zero_staging():
        zero_rowsf()
        for j in range(RSL // (CHUNK * EMBED_DIM)):
            pltpu.sync_copy(
                rowsf,
                srows.at[pl.ds(sid * RSL + j * CHUNK * EMBED_DIM,
                               CHUNK * EMBED_DIM)])
        pltpu.sync_copy(rowsf.at[pl.ds(0, BSL)],
                        sbias.at[pl.ds(sid * BSL, BSL)])

    def copy_out(rows_out, bias_out):
        pltpu.sync_copy(srows.at[pl.ds(sid * RSL, RSL)],
                        rows_out.at[pl.ds(core * BATCH * EMBED_DIM
                                          + sid * RSL, RSL)])
        pltpu.sync_copy(sbias.at[pl.ds(sid * BSL, BSL)],
                        bias_out.at[pl.ds(core * BATCH + sid * BSL, BSL)])

    zero_staging()
    plsc.subcore_barrier()
    process_table(uidx_hbm, ue_hbm, ub_hbm)
    plsc.subcore_barrier()
    copy_out(su_hbm, sub_hbm)
    zero_staging()
    plsc.subcore_barrier()
    process_table(midx_hbm, me_hbm, mb_hbm)
    plsc.subcore_barrier()
    copy_out(sm_hbm, smb_hbm)


def _k2_body(su_ref, sm_ref, sub_ref, smb_ref, out_ref):
    u = su_ref[0] + su_ref[1]
    m = sm_ref[0] + sm_ref[1]
    dot = jnp.sum(u * m, axis=1)
    x = dot + sub_ref[0] + sub_ref[1] + smb_ref[0] + smb_ref[1]
    r = jax.nn.sigmoid(x)
    out_ref[...] = r * (MAX_RATING - MIN_RATING) + MIN_RATING


_K2_BLK = 2048


def _k2(su, sm, sub, smb):
    grid = (BATCH // _K2_BLK,)
    return pl.pallas_call(
        _k2_body,
        out_shape=jax.ShapeDtypeStruct((BATCH,), jnp.float32),
        grid=grid,
        in_specs=[
            pl.BlockSpec((2, _K2_BLK, EMBED_DIM), lambda i: (0, i, 0)),
            pl.BlockSpec((2, _K2_BLK, EMBED_DIM), lambda i: (0, i, 0)),
            pl.BlockSpec((2, _K2_BLK), lambda i: (0, i)),
            pl.BlockSpec((2, _K2_BLK), lambda i: (0, i)),
        ],
        out_specs=pl.BlockSpec((_K2_BLK,), lambda i: (i,)),
    )(su.reshape(2, BATCH, EMBED_DIM), sm.reshape(2, BATCH, EMBED_DIM),
      sub.reshape(2, BATCH), smb.reshape(2, BATCH))


def kernel(inputs, user_embedding, user_bias, movie_embedding, movie_bias):
    idx = inputs.astype(jnp.int32)
    su, sm, sub, smb = _k1(idx[:, 0], idx[:, 1],
                           user_embedding.T, movie_embedding.T,
                           user_bias.reshape(-1), movie_bias.reshape(-1))
    return _k2(su, sm, sub, smb)
